# baseline (speedup 1.0000x reference)
import functools

import jax
import jax.numpy as jnp
from jax import lax
from jax.experimental import pallas as pl
from jax.experimental.pallas import tpu as pltpu
from jax.experimental.pallas import tpu_sc as plsc

CAP = 1000000
XD = 32
YD = 32
B = 16384

NC = 2
NS = 16
NW = NC * NS
BPW = B // NW
IDX_CHUNK = 128
NIC = BPW // IDX_CHUNK


def _update_body(pos_ref, x_ref, y3_ref, xbuf_any, ybuf3_any, yvar2_any,
                 newx, newy3, newyvar2, var_scratch, sem1, sem2, sem3):
    pos = pos_ref[0]
    pos8 = pos // 8

    c1 = pltpu.make_async_copy(x_ref, newx.at[pl.ds(pos, B), :], sem1)
    c1.start()

    y3 = y3_ref[...]
    s = jnp.sum(y3, axis=2)
    ss = jnp.sum(y3 * y3, axis=2)
    var_scratch[...] = (ss - s * s * (1.0 / YD)) * (1.0 / (YD - 1))

    c2 = pltpu.make_async_copy(y3_ref, newy3.at[pl.ds(pos8, B // 8), :, :], sem2)
    c2.start()
    c3 = pltpu.make_async_copy(var_scratch, newyvar2.at[pl.ds(pos8, B // 8), :], sem3)
    c3.start()
    c1.wait()
    c2.wait()
    c3.wait()


def _push_update(pos_arr, x, y3, x_buffer, y_buffer3, y_var2):
    return pl.pallas_call(
        _update_body,
        in_specs=[
            pl.BlockSpec(memory_space=pltpu.SMEM),
            pl.BlockSpec(memory_space=pltpu.VMEM),
            pl.BlockSpec(memory_space=pltpu.VMEM),
            pl.BlockSpec(memory_space=pltpu.MemorySpace.HBM),
            pl.BlockSpec(memory_space=pltpu.MemorySpace.HBM),
            pl.BlockSpec(memory_space=pltpu.MemorySpace.HBM),
        ],
        out_specs=[
            pl.BlockSpec(memory_space=pltpu.MemorySpace.HBM),
            pl.BlockSpec(memory_space=pltpu.MemorySpace.HBM),
            pl.BlockSpec(memory_space=pltpu.MemorySpace.HBM),
        ],
        out_shape=[
            jax.ShapeDtypeStruct((CAP, XD), jnp.float32),
            jax.ShapeDtypeStruct((CAP // 8, 8, YD), jnp.float32),
            jax.ShapeDtypeStruct((CAP // 8, 8), jnp.float32),
        ],
        scratch_shapes=[
            pltpu.VMEM((B // 8, 8), jnp.float32),
            pltpu.SemaphoreType.DMA,
            pltpu.SemaphoreType.DMA,
            pltpu.SemaphoreType.DMA,
        ],
        input_output_aliases={3: 0, 4: 1, 5: 2},
    )(pos_arr, x, y3, x_buffer, y_buffer3, y_var2)


def _gather_body(newx_hbm, newy_hbm, idx3_hbm, out_hbm,
                 idx_v, gx_v, gy_v, semx, semy):
    wid = lax.axis_index("s") * NC + lax.axis_index("c")
    base = wid * BPW

    pltpu.sync_copy(idx3_hbm.at[wid], idx_v)

    copies = []
    for k in range(NIC):
        copies.append(pltpu.async_copy(
            newx_hbm.at[idx_v.at[k]],
            gx_v.at[pl.ds(k * IDX_CHUNK, IDX_CHUNK), :], semx))
        copies.append(pltpu.async_copy(
            newy_hbm.at[idx_v.at[k]],
            gy_v.at[pl.ds(k * IDX_CHUNK, IDX_CHUNK), :], semy))
    for c in copies:
        c.wait()

    pltpu.sync_copy(gx_v, out_hbm.at[pl.ds(base, BPW), pl.ds(0, XD)])
    pltpu.sync_copy(gy_v, out_hbm.at[pl.ds(base, BPW), pl.ds(XD, YD)])


def _sample_gather(newx, newy, idx3):
    mesh = plsc.VectorSubcoreMesh(core_axis_name="c", subcore_axis_name="s")
    return pl.kernel(
        _gather_body,
        out_type=jax.ShapeDtypeStruct((B, XD + YD), jnp.float32),
        mesh=mesh,
        compiler_params=pltpu.CompilerParams(use_tc_tiling_on_sc=False),
        scratch_types=[
            pltpu.VMEM((NIC, IDX_CHUNK), jnp.int32),
            pltpu.VMEM((BPW, XD), jnp.float32),
            pltpu.VMEM((BPW, YD), jnp.float32),
            pltpu.SemaphoreType.DMA,
            pltpu.SemaphoreType.DMA,
        ],
    )(newx, newy, idx3)


def kernel(x_buffer, y_buffer, y_var_buffer, x, y, position, indices):
    pos_arr = jnp.asarray(position, jnp.int32).reshape(1)
    y3 = y.reshape(B // 8, 8, YD)
    y_buffer3 = y_buffer.reshape(CAP // 8, 8, YD)
    y_var2 = y_var_buffer.reshape(CAP // 8, 8)
    idx3 = indices.reshape(NW, NIC, IDX_CHUNK)

    new_x_buffer, new_y_buffer3, new_y_var2 = _push_update(
        pos_arr, x, y3, x_buffer, y_buffer3, y_var2)
    new_y_buffer = new_y_buffer3.reshape(CAP, YD)
    new_y_var_buffer = new_y_var2.reshape(CAP)

    out = _sample_gather(new_x_buffer, new_y_buffer, idx3)
    return (out, new_x_buffer, new_y_buffer, new_y_var_buffer)
